# Initial kernel scaffold; baseline (speedup 1.0000x reference)
import jax
import jax.numpy as jnp
from jax.experimental import pallas as pl


def kernel(x, edge_index, batch, W1_0, b1_0, W1_1, b1_1, W2_0, b2_0, W2_1, b2_1, W3_0, b3_0, W3_1, b3_1, Wf0, bf0, Wf1, bf1):
    raise NotImplementedError("write your pallas kernel here")


# speedup vs baseline: 22.2600x; 22.2600x over previous
import functools

import jax
import jax.numpy as jnp
from jax import lax
from jax.experimental import pallas as pl
from jax.experimental.pallas import tpu as pltpu
from jax.experimental.pallas import tpu_sc as plsc

N = 100000
E = 6400000
G_PAD = 1024

_NC, _NS, _L = 2, 16, 16
_CH = 128
_GRP = 8
_NCHUNK = 3128
_NIT = _NCHUNK // _GRP
_EP = _NS * _NCHUNK * _CH
_NACC = N + 16
_RPW = 6256
_RPW_LAST = N - (_NS - 1) * _RPW


def _sc_mesh():
    return plsc.VectorSubcoreMesh(
        core_axis_name="c", subcore_axis_name="s",
        num_cores=_NC, num_subcores=_NS)


_SC_PARAMS = pltpu.CompilerParams(use_tc_tiling_on_sc=False)


def _sc_scatter_body(hp_hbm, src_hbm, dst_hbm, zeros_hbm, agg_hbm,
                     idx_s, idx_d, msg, acc, sem):
    c = lax.axis_index("c")
    s = lax.axis_index("s")

    @pl.when(s < _NS - 1)
    def _():
        pltpu.sync_copy(zeros_hbm, acc.at[pl.ds(s * _RPW, _RPW)])

    @pl.when(s == _NS - 1)
    def _():
        pltpu.sync_copy(zeros_hbm.at[pl.ds(0, _RPW_LAST)],
                        acc.at[pl.ds((_NS - 1) * _RPW, _RPW_LAST)])

    plsc.subcore_barrier()

    table = hp_hbm.at[c]

    @pl.loop(0, _NIT)
    def _(it):
        base = it * _GRP
        pltpu.sync_copy(src_hbm.at[s, pl.ds(base, _GRP)], idx_s)
        pltpu.sync_copy(dst_hbm.at[s, pl.ds(base, _GRP)], idx_d)
        cps = [pltpu.async_copy(table.at[idx_s.at[b]], msg.at[b], sem)
               for b in range(_GRP)]
        for cp in cps:
            cp.wait()
        for b in range(_GRP):
            pltpu.sync_copy(msg.at[b], acc.at[idx_d.at[b]], add=True)

    plsc.subcore_barrier()

    @pl.when(s < _NS - 1)
    def _():
        pltpu.sync_copy(acc.at[pl.ds(s * _RPW, _RPW)],
                        agg_hbm.at[c, pl.ds(s * _RPW, _RPW)])

    @pl.when(s == _NS - 1)
    def _():
        pltpu.sync_copy(acc.at[pl.ds((_NS - 1) * _RPW, _RPW_LAST)],
                        agg_hbm.at[c, pl.ds((_NS - 1) * _RPW, _RPW_LAST)])


def _sc_scatter(hp, src3, dst3, zeros):
    k = pl.kernel(
        _sc_scatter_body,
        out_type=jax.ShapeDtypeStruct((_NC, N, _L), jnp.float32),
        mesh=_sc_mesh(),
        scratch_types=[
            pltpu.VMEM((_GRP, _CH), jnp.int32),
            pltpu.VMEM((_GRP, _CH), jnp.int32),
            pltpu.VMEM((_GRP, _CH, _L), jnp.float32),
            pltpu.VMEM_SHARED((_NACC, _L), jnp.float32),
            pltpu.SemaphoreType.DMA,
        ],
        compiler_params=_SC_PARAMS,
    )
    return k(hp, src3, dst3, zeros)


def _sc_head_gather_body(xp_hbm, pos_hbm, g_hbm, pidx, rows, sem):
    c = lax.axis_index("c")
    s = lax.axis_index("s")
    per_w = G_PAD // _NS
    pltpu.sync_copy(pos_hbm.at[pl.ds(s * per_w, per_w)], pidx)
    pltpu.async_copy(xp_hbm.at[c].at[pidx], rows, sem).wait()
    pltpu.sync_copy(rows, g_hbm.at[c, pl.ds(s * per_w, per_w)])


def _sc_head_gather(xp, pos):
    per_w = G_PAD // _NS
    k = pl.kernel(
        _sc_head_gather_body,
        out_type=jax.ShapeDtypeStruct((_NC, G_PAD, _L), jnp.float32),
        mesh=_sc_mesh(),
        scratch_types=[
            pltpu.VMEM((per_w,), jnp.int32),
            pltpu.VMEM((per_w, _L), jnp.float32),
            pltpu.SemaphoreType.DMA,
        ],
        compiler_params=_SC_PARAMS,
    )
    return k(xp, pos)


_RB = 2000


def _pack(h):
    lo = h[:, :16]
    hi = jnp.concatenate([h[:, 16:19], jnp.zeros((h.shape[0], 13), h.dtype)],
                         axis=1)
    return lo, hi


def _k0_body(x_ref, w_ref, b_ref, out_ref):
    h = jnp.dot(x_ref[...], w_ref[...],
                preferred_element_type=jnp.float32) + b_ref[...]
    lo, hi = _pack(h)
    out_ref[0] = lo
    out_ref[1] = hi


def _k0(x, W, b):
    return pl.pallas_call(
        _k0_body,
        grid=(N // _RB,),
        in_specs=[
            pl.BlockSpec((_RB, 11), lambda i: (i, 0)),
            pl.BlockSpec((11, 19), lambda i: (0, 0)),
            pl.BlockSpec((1, 19), lambda i: (0, 0)),
        ],
        out_specs=pl.BlockSpec((2, _RB, 16), lambda i: (0, i, 0)),
        out_shape=jax.ShapeDtypeStruct((2, N, 16), jnp.float32),
    )(x, W, b.reshape(1, 19))


def _mid_body(has_residual, fin, x_ref, agg_ref, w1_ref, b1_ref,
              w0n_ref, b0n_ref, x1_ref, hp_ref):
    a19 = jnp.concatenate([agg_ref[0], agg_ref[1][:, :3]], axis=1)
    cat = jnp.concatenate([x_ref[...], a19], axis=1)
    y = jnp.dot(cat, w1_ref[...], preferred_element_type=jnp.float32) \
        + b1_ref[...]
    if has_residual:
        y = y + x_ref[...][:, :19]
    y = jnp.maximum(y, 0.0)
    x1_ref[...] = y
    h = jnp.dot(y, w0n_ref[...], preferred_element_type=jnp.float32) \
        + b0n_ref[...]
    lo, hi = _pack(h)
    hp_ref[0] = lo
    hp_ref[1] = hi


def _mid(x, agg, W1, b1, W0n, b0n, has_residual):
    fin = x.shape[1]
    body = functools.partial(_mid_body, has_residual, fin)
    return pl.pallas_call(
        body,
        grid=(N // _RB,),
        in_specs=[
            pl.BlockSpec((_RB, fin), lambda i: (i, 0)),
            pl.BlockSpec((2, _RB, 16), lambda i: (0, i, 0)),
            pl.BlockSpec((fin + 19, 19), lambda i: (0, 0)),
            pl.BlockSpec((1, 19), lambda i: (0, 0)),
            pl.BlockSpec((19, 19), lambda i: (0, 0)),
            pl.BlockSpec((1, 19), lambda i: (0, 0)),
        ],
        out_specs=[
            pl.BlockSpec((_RB, 19), lambda i: (i, 0)),
            pl.BlockSpec((2, _RB, 16), lambda i: (0, i, 0)),
        ],
        out_shape=[
            jax.ShapeDtypeStruct((N, 19), jnp.float32),
            jax.ShapeDtypeStruct((2, N, 16), jnp.float32),
        ],
    )(x, agg, W1, b1.reshape(1, 19), W0n, b0n.reshape(1, 19))


def _k3_body(x_ref, agg_ref, w1_ref, b1_ref, out_ref):
    a19 = jnp.concatenate([agg_ref[0], agg_ref[1][:, :3]], axis=1)
    cat = jnp.concatenate([x_ref[...], a19], axis=1)
    y = jnp.dot(cat, w1_ref[...], preferred_element_type=jnp.float32) \
        + b1_ref[...]
    y = jnp.maximum(y + x_ref[...], 0.0)
    lo, hi = _pack(y)
    out_ref[0] = lo
    out_ref[1] = hi


def _k3(x, agg, W1, b1):
    return pl.pallas_call(
        _k3_body,
        grid=(N // _RB,),
        in_specs=[
            pl.BlockSpec((_RB, 19), lambda i: (i, 0)),
            pl.BlockSpec((2, _RB, 16), lambda i: (0, i, 0)),
            pl.BlockSpec((38, 19), lambda i: (0, 0)),
            pl.BlockSpec((1, 19), lambda i: (0, 0)),
        ],
        out_specs=pl.BlockSpec((2, _RB, 16), lambda i: (0, i, 0)),
        out_shape=jax.ShapeDtypeStruct((2, N, 16), jnp.float32),
    )(x, agg, W1, b1.reshape(1, 19))


_PB = 2000


def _pos_body(batch_ref, out_ref):
    i = pl.program_id(0)

    @pl.when(i == 0)
    def _():
        out_ref[...] = jnp.zeros_like(out_ref)

    b = batch_ref[0]
    gids = lax.broadcasted_iota(jnp.int32, (G_PAD, 1), 0)
    le = (b <= gids).astype(jnp.int32)
    out_ref[...] += jnp.sum(le, axis=1, keepdims=True)

    @pl.when(i == (N // _PB) - 1)
    def _():
        out_ref[...] = jnp.maximum(out_ref[...] - 1, 0)


def _pos_kernel(batch):
    out = pl.pallas_call(
        _pos_body,
        grid=(N // _PB,),
        in_specs=[pl.BlockSpec((1, 1, _PB), lambda i: (i, 0, 0))],
        out_specs=pl.BlockSpec((G_PAD, 1), lambda i: (0, 0)),
        out_shape=jax.ShapeDtypeStruct((G_PAD, 1), jnp.int32),
    )(batch.reshape(N // _PB, 1, _PB))
    return out.reshape(G_PAD)


def _head_body(g_ref, w0_ref, b0_ref, w1_ref, b1_ref, out_ref):
    g = jnp.concatenate([g_ref[0, :1000, :], g_ref[1, :1000, :3]], axis=1)
    t = jnp.maximum(
        jnp.dot(g, w0_ref[...], preferred_element_type=jnp.float32)
        + b0_ref[...], 0.0)
    out_ref[...] = jnp.maximum(
        jnp.dot(t, w1_ref[...], preferred_element_type=jnp.float32)
        + b1_ref[...], 0.0)


def _head(g2, Wf0, bf0, Wf1, bf1):
    return pl.pallas_call(
        _head_body,
        in_specs=[
            pl.BlockSpec((2, G_PAD, 16), lambda: (0, 0, 0)),
            pl.BlockSpec((19, 10), lambda: (0, 0)),
            pl.BlockSpec((1, 10), lambda: (0, 0)),
            pl.BlockSpec((10, 1), lambda: (0, 0)),
            pl.BlockSpec((1, 1), lambda: (0, 0)),
        ],
        out_specs=pl.BlockSpec((1000, 1), lambda: (0, 0)),
        out_shape=jax.ShapeDtypeStruct((1000, 1), jnp.float32),
        grid=(),
    )(g2, Wf0, bf0.reshape(1, 10), Wf1, bf1.reshape(1, 1))


def kernel(x, edge_index, batch, W1_0, b1_0, W1_1, b1_1, W2_0, b2_0,
           W2_1, b2_1, W3_0, b3_0, W3_1, b3_1, Wf0, bf0, Wf1, bf1):
    pad = _EP - E
    src3 = jnp.concatenate(
        [edge_index[0], jnp.zeros((pad,), jnp.int32)]).reshape(
            _NS, _NCHUNK, _CH)
    dst3 = jnp.concatenate(
        [edge_index[1], jnp.full((pad,), N, jnp.int32)]).reshape(
            _NS, _NCHUNK, _CH)
    zeros = jnp.zeros((_RPW, _L), jnp.float32)

    hp1 = _k0(x, W1_0, b1_0)
    agg1 = _sc_scatter(hp1, src3, dst3, zeros)
    x1, hp2 = _mid(x, agg1, W1_1, b1_1, W2_0, b2_0, has_residual=False)
    agg2 = _sc_scatter(hp2, src3, dst3, zeros)
    x2, hp3 = _mid(x1, agg2, W2_1, b2_1, W3_0, b3_0, has_residual=True)
    agg3 = _sc_scatter(hp3, src3, dst3, zeros)
    x3p = _k3(x2, agg3, W3_1, b3_1)

    pos = _pos_kernel(batch)
    g2 = _sc_head_gather(x3p, pos)
    return _head(g2, Wf0, bf0, Wf1, bf1)


# speedup vs baseline: 27.8943x; 1.2531x over previous
import functools

import jax
import jax.numpy as jnp
from jax import lax
from jax.experimental import pallas as pl
from jax.experimental.pallas import tpu as pltpu
from jax.experimental.pallas import tpu_sc as plsc

N = 100000
E = 6400000
G_PAD = 1024

_NC, _NS, _L = 2, 16, 16
_CH = 128
_GRP = 4
_NCHUNK = 3136
_NIT = _NCHUNK // _GRP
_EP = _NS * _NCHUNK * _CH
_NACC = N + 16
_RPW = 6256
_RPW_LAST = N - (_NS - 1) * _RPW


def _sc_mesh():
    return plsc.VectorSubcoreMesh(
        core_axis_name="c", subcore_axis_name="s",
        num_cores=_NC, num_subcores=_NS)


_SC_PARAMS = pltpu.CompilerParams(use_tc_tiling_on_sc=False)


def _sc_scatter_body(hp_hbm, src_hbm, dst_hbm, zeros_hbm, agg_hbm,
                     idx_s, idx_d, msg, acc, gsem, ssem):
    c = lax.axis_index("c")
    s = lax.axis_index("s")

    @pl.when(s < _NS - 1)
    def _():
        pltpu.sync_copy(zeros_hbm, acc.at[pl.ds(s * _RPW, _RPW)])

    @pl.when(s == _NS - 1)
    def _():
        pltpu.sync_copy(zeros_hbm.at[pl.ds(0, _RPW_LAST)],
                        acc.at[pl.ds((_NS - 1) * _RPW, _RPW_LAST)])

    plsc.subcore_barrier()

    table = hp_hbm.at[c]

    def _load_idx(g, q):
        pltpu.sync_copy(src_hbm.at[s, pl.ds(g * _GRP, _GRP)], idx_s.at[q])
        pltpu.sync_copy(dst_hbm.at[s, pl.ds(g * _GRP, _GRP)], idx_d.at[q])

    def _fire_gathers(q):
        for b in range(_GRP):
            pltpu.async_copy(table.at[idx_s.at[q, b]], msg.at[q, b],
                             gsem.at[q])

    def _wait_gathers(q):
        for b in range(_GRP):
            pltpu.make_async_copy(table.at[idx_s.at[q, b]], msg.at[q, b],
                                  gsem.at[q]).wait()

    def _fire_scatters(q):
        for b in range(_GRP):
            pltpu.async_copy(msg.at[q, b], acc.at[idx_d.at[q, b]],
                             ssem.at[q], add=True)

    def _wait_scatters(q):
        for b in range(_GRP):
            pltpu.make_async_copy(msg.at[q, b], acc.at[idx_d.at[q, b]],
                                  ssem.at[q]).wait()

    _load_idx(0, 0)
    _fire_gathers(0)

    @pl.loop(0, (_NIT - 2) // 2)
    def _(t):
        for q in (0, 1):
            g = t * 2 + q

            @pl.when(g >= 1)
            def _():
                _wait_scatters(1 - q)

            _load_idx(g + 1, 1 - q)
            _fire_gathers(1 - q)
            _wait_gathers(q)
            _fire_scatters(q)

    _wait_scatters(1)
    _load_idx(_NIT - 1, 1)
    _fire_gathers(1)
    _wait_gathers(0)
    _fire_scatters(0)
    _wait_gathers(1)
    _fire_scatters(1)
    _wait_scatters(0)
    _wait_scatters(1)

    plsc.subcore_barrier()

    @pl.when(s < _NS - 1)
    def _():
        pltpu.sync_copy(acc.at[pl.ds(s * _RPW, _RPW)],
                        agg_hbm.at[c, pl.ds(s * _RPW, _RPW)])

    @pl.when(s == _NS - 1)
    def _():
        pltpu.sync_copy(acc.at[pl.ds((_NS - 1) * _RPW, _RPW_LAST)],
                        agg_hbm.at[c, pl.ds((_NS - 1) * _RPW, _RPW_LAST)])


def _sc_scatter(hp, src3, dst3, zeros):
    k = pl.kernel(
        _sc_scatter_body,
        out_type=jax.ShapeDtypeStruct((_NC, N, _L), jnp.float32),
        mesh=_sc_mesh(),
        scratch_types=[
            pltpu.VMEM((2, _GRP, _CH), jnp.int32),
            pltpu.VMEM((2, _GRP, _CH), jnp.int32),
            pltpu.VMEM((2, _GRP, _CH, _L), jnp.float32),
            pltpu.VMEM_SHARED((_NACC, _L), jnp.float32),
            pltpu.SemaphoreType.DMA((2,)),
            pltpu.SemaphoreType.DMA((2,)),
        ],
        compiler_params=_SC_PARAMS,
    )
    return k(hp, src3, dst3, zeros)


def _sc_head_gather_body(xp_hbm, pos_hbm, g_hbm, pidx, rows, sem):
    c = lax.axis_index("c")
    s = lax.axis_index("s")
    per_w = G_PAD // _NS
    pltpu.sync_copy(pos_hbm.at[pl.ds(s * per_w, per_w)], pidx)
    pltpu.async_copy(xp_hbm.at[c].at[pidx], rows, sem).wait()
    pltpu.sync_copy(rows, g_hbm.at[c, pl.ds(s * per_w, per_w)])


def _sc_head_gather(xp, pos):
    per_w = G_PAD // _NS
    k = pl.kernel(
        _sc_head_gather_body,
        out_type=jax.ShapeDtypeStruct((_NC, G_PAD, _L), jnp.float32),
        mesh=_sc_mesh(),
        scratch_types=[
            pltpu.VMEM((per_w,), jnp.int32),
            pltpu.VMEM((per_w, _L), jnp.float32),
            pltpu.SemaphoreType.DMA,
        ],
        compiler_params=_SC_PARAMS,
    )
    return k(xp, pos)


_RB = 2000


def _pack(h):
    lo = h[:, :16]
    hi = jnp.concatenate([h[:, 16:19], jnp.zeros((h.shape[0], 13), h.dtype)],
                         axis=1)
    return lo, hi


def _k0_body(x_ref, w_ref, b_ref, out_ref):
    h = jnp.dot(x_ref[...], w_ref[...],
                preferred_element_type=jnp.float32) + b_ref[...]
    lo, hi = _pack(h)
    out_ref[0] = lo
    out_ref[1] = hi


def _k0(x, W, b):
    return pl.pallas_call(
        _k0_body,
        grid=(N // _RB,),
        in_specs=[
            pl.BlockSpec((_RB, 11), lambda i: (i, 0)),
            pl.BlockSpec((11, 19), lambda i: (0, 0)),
            pl.BlockSpec((1, 19), lambda i: (0, 0)),
        ],
        out_specs=pl.BlockSpec((2, _RB, 16), lambda i: (0, i, 0)),
        out_shape=jax.ShapeDtypeStruct((2, N, 16), jnp.float32),
    )(x, W, b.reshape(1, 19))


def _mid_body(has_residual, fin, x_ref, agg_ref, w1_ref, b1_ref,
              w0n_ref, b0n_ref, x1_ref, hp_ref):
    a19 = jnp.concatenate([agg_ref[0], agg_ref[1][:, :3]], axis=1)
    cat = jnp.concatenate([x_ref[...], a19], axis=1)
    y = jnp.dot(cat, w1_ref[...], preferred_element_type=jnp.float32) \
        + b1_ref[...]
    if has_residual:
        y = y + x_ref[...][:, :19]
    y = jnp.maximum(y, 0.0)
    x1_ref[...] = y
    h = jnp.dot(y, w0n_ref[...], preferred_element_type=jnp.float32) \
        + b0n_ref[...]
    lo, hi = _pack(h)
    hp_ref[0] = lo
    hp_ref[1] = hi


def _mid(x, agg, W1, b1, W0n, b0n, has_residual):
    fin = x.shape[1]
    body = functools.partial(_mid_body, has_residual, fin)
    return pl.pallas_call(
        body,
        grid=(N // _RB,),
        in_specs=[
            pl.BlockSpec((_RB, fin), lambda i: (i, 0)),
            pl.BlockSpec((2, _RB, 16), lambda i: (0, i, 0)),
            pl.BlockSpec((fin + 19, 19), lambda i: (0, 0)),
            pl.BlockSpec((1, 19), lambda i: (0, 0)),
            pl.BlockSpec((19, 19), lambda i: (0, 0)),
            pl.BlockSpec((1, 19), lambda i: (0, 0)),
        ],
        out_specs=[
            pl.BlockSpec((_RB, 19), lambda i: (i, 0)),
            pl.BlockSpec((2, _RB, 16), lambda i: (0, i, 0)),
        ],
        out_shape=[
            jax.ShapeDtypeStruct((N, 19), jnp.float32),
            jax.ShapeDtypeStruct((2, N, 16), jnp.float32),
        ],
    )(x, agg, W1, b1.reshape(1, 19), W0n, b0n.reshape(1, 19))


def _k3_body(x_ref, agg_ref, w1_ref, b1_ref, out_ref):
    a19 = jnp.concatenate([agg_ref[0], agg_ref[1][:, :3]], axis=1)
    cat = jnp.concatenate([x_ref[...], a19], axis=1)
    y = jnp.dot(cat, w1_ref[...], preferred_element_type=jnp.float32) \
        + b1_ref[...]
    y = jnp.maximum(y + x_ref[...], 0.0)
    lo, hi = _pack(y)
    out_ref[0] = lo
    out_ref[1] = hi


def _k3(x, agg, W1, b1):
    return pl.pallas_call(
        _k3_body,
        grid=(N // _RB,),
        in_specs=[
            pl.BlockSpec((_RB, 19), lambda i: (i, 0)),
            pl.BlockSpec((2, _RB, 16), lambda i: (0, i, 0)),
            pl.BlockSpec((38, 19), lambda i: (0, 0)),
            pl.BlockSpec((1, 19), lambda i: (0, 0)),
        ],
        out_specs=pl.BlockSpec((2, _RB, 16), lambda i: (0, i, 0)),
        out_shape=jax.ShapeDtypeStruct((2, N, 16), jnp.float32),
    )(x, agg, W1, b1.reshape(1, 19))


_PB = 2000


def _pos_body(batch_ref, out_ref):
    i = pl.program_id(0)

    @pl.when(i == 0)
    def _():
        out_ref[...] = jnp.zeros_like(out_ref)

    b = batch_ref[0]
    gids = lax.broadcasted_iota(jnp.int32, (G_PAD, 1), 0)
    le = (b <= gids).astype(jnp.int32)
    out_ref[...] += jnp.sum(le, axis=1, keepdims=True)

    @pl.when(i == (N // _PB) - 1)
    def _():
        out_ref[...] = jnp.maximum(out_ref[...] - 1, 0)


def _pos_kernel(batch):
    out = pl.pallas_call(
        _pos_body,
        grid=(N // _PB,),
        in_specs=[pl.BlockSpec((1, 1, _PB), lambda i: (i, 0, 0))],
        out_specs=pl.BlockSpec((G_PAD, 1), lambda i: (0, 0)),
        out_shape=jax.ShapeDtypeStruct((G_PAD, 1), jnp.int32),
    )(batch.reshape(N // _PB, 1, _PB))
    return out.reshape(G_PAD)


def _head_body(g_ref, w0_ref, b0_ref, w1_ref, b1_ref, out_ref):
    g = jnp.concatenate([g_ref[0, :1000, :], g_ref[1, :1000, :3]], axis=1)
    t = jnp.maximum(
        jnp.dot(g, w0_ref[...], preferred_element_type=jnp.float32)
        + b0_ref[...], 0.0)
    out_ref[...] = jnp.maximum(
        jnp.dot(t, w1_ref[...], preferred_element_type=jnp.float32)
        + b1_ref[...], 0.0)


def _head(g2, Wf0, bf0, Wf1, bf1):
    return pl.pallas_call(
        _head_body,
        in_specs=[
            pl.BlockSpec((2, G_PAD, 16), lambda: (0, 0, 0)),
            pl.BlockSpec((19, 10), lambda: (0, 0)),
            pl.BlockSpec((1, 10), lambda: (0, 0)),
            pl.BlockSpec((10, 1), lambda: (0, 0)),
            pl.BlockSpec((1, 1), lambda: (0, 0)),
        ],
        out_specs=pl.BlockSpec((1000, 1), lambda: (0, 0)),
        out_shape=jax.ShapeDtypeStruct((1000, 1), jnp.float32),
        grid=(),
    )(g2, Wf0, bf0.reshape(1, 10), Wf1, bf1.reshape(1, 1))


def kernel(x, edge_index, batch, W1_0, b1_0, W1_1, b1_1, W2_0, b2_0,
           W2_1, b2_1, W3_0, b3_0, W3_1, b3_1, Wf0, bf0, Wf1, bf1):
    pad = _EP - E
    src3 = jnp.concatenate(
        [edge_index[0], jnp.zeros((pad,), jnp.int32)]).reshape(
            _NS, _NCHUNK, _CH)
    dst3 = jnp.concatenate(
        [edge_index[1], jnp.full((pad,), N, jnp.int32)]).reshape(
            _NS, _NCHUNK, _CH)
    zeros = jnp.zeros((_RPW, _L), jnp.float32)

    hp1 = _k0(x, W1_0, b1_0)
    agg1 = _sc_scatter(hp1, src3, dst3, zeros)
    x1, hp2 = _mid(x, agg1, W1_1, b1_1, W2_0, b2_0, has_residual=False)
    agg2 = _sc_scatter(hp2, src3, dst3, zeros)
    x2, hp3 = _mid(x1, agg2, W2_1, b2_1, W3_0, b3_0, has_residual=True)
    agg3 = _sc_scatter(hp3, src3, dst3, zeros)
    x3p = _k3(x2, agg3, W3_1, b3_1)

    pos = _pos_kernel(batch)
    g2 = _sc_head_gather(x3p, pos)
    return _head(g2, Wf0, bf0, Wf1, bf1)


# speedup vs baseline: 28.0431x; 1.0053x over previous
import functools

import jax
import jax.numpy as jnp
from jax import lax
from jax.experimental import pallas as pl
from jax.experimental.pallas import tpu as pltpu
from jax.experimental.pallas import tpu_sc as plsc

N = 100000
E = 6400000
G_PAD = 1024

_NC, _NS, _L = 2, 16, 16
_CH = 512
_GRP = 1
_NCHUNK = 784
_NIT = _NCHUNK // _GRP
_EP = _NS * _NCHUNK * _CH
_NACC = N + 16
_RPW = 6256
_RPW_LAST = N - (_NS - 1) * _RPW


def _sc_mesh():
    return plsc.VectorSubcoreMesh(
        core_axis_name="c", subcore_axis_name="s",
        num_cores=_NC, num_subcores=_NS)


_SC_PARAMS = pltpu.CompilerParams(use_tc_tiling_on_sc=False)


def _sc_scatter_body(hp_hbm, src_hbm, dst_hbm, zeros_hbm, agg_hbm,
                     idx_s, idx_d, msg, acc, gsem, ssem):
    c = lax.axis_index("c")
    s = lax.axis_index("s")

    @pl.when(s < _NS - 1)
    def _():
        pltpu.sync_copy(zeros_hbm, acc.at[pl.ds(s * _RPW, _RPW)])

    @pl.when(s == _NS - 1)
    def _():
        pltpu.sync_copy(zeros_hbm.at[pl.ds(0, _RPW_LAST)],
                        acc.at[pl.ds((_NS - 1) * _RPW, _RPW_LAST)])

    plsc.subcore_barrier()

    table = hp_hbm.at[c]

    def _load_idx(g, q):
        pltpu.sync_copy(src_hbm.at[s, pl.ds(g * _GRP, _GRP)], idx_s.at[q])
        pltpu.sync_copy(dst_hbm.at[s, pl.ds(g * _GRP, _GRP)], idx_d.at[q])

    def _fire_gathers(q):
        for b in range(_GRP):
            pltpu.async_copy(table.at[idx_s.at[q, b]], msg.at[q, b],
                             gsem.at[q])

    def _wait_gathers(q):
        for b in range(_GRP):
            pltpu.make_async_copy(table.at[idx_s.at[q, b]], msg.at[q, b],
                                  gsem.at[q]).wait()

    def _fire_scatters(q):
        for b in range(_GRP):
            pltpu.async_copy(msg.at[q, b], acc.at[idx_d.at[q, b]],
                             ssem.at[q], add=True)

    def _wait_scatters(q):
        for b in range(_GRP):
            pltpu.make_async_copy(msg.at[q, b], acc.at[idx_d.at[q, b]],
                                  ssem.at[q]).wait()

    _load_idx(0, 0)
    _fire_gathers(0)

    @pl.loop(0, (_NIT - 2) // 2)
    def _(t):
        for q in (0, 1):
            g = t * 2 + q

            @pl.when(g >= 1)
            def _():
                _wait_scatters(1 - q)

            _load_idx(g + 1, 1 - q)
            _fire_gathers(1 - q)
            _wait_gathers(q)
            _fire_scatters(q)

    _wait_scatters(1)
    _load_idx(_NIT - 1, 1)
    _fire_gathers(1)
    _wait_gathers(0)
    _fire_scatters(0)
    _wait_gathers(1)
    _fire_scatters(1)
    _wait_scatters(0)
    _wait_scatters(1)

    plsc.subcore_barrier()

    @pl.when(s < _NS - 1)
    def _():
        pltpu.sync_copy(acc.at[pl.ds(s * _RPW, _RPW)],
                        agg_hbm.at[c, pl.ds(s * _RPW, _RPW)])

    @pl.when(s == _NS - 1)
    def _():
        pltpu.sync_copy(acc.at[pl.ds((_NS - 1) * _RPW, _RPW_LAST)],
                        agg_hbm.at[c, pl.ds((_NS - 1) * _RPW, _RPW_LAST)])


def _sc_scatter(hp, src3, dst3, zeros):
    k = pl.kernel(
        _sc_scatter_body,
        out_type=jax.ShapeDtypeStruct((_NC, N, _L), jnp.float32),
        mesh=_sc_mesh(),
        scratch_types=[
            pltpu.VMEM((2, _GRP, _CH), jnp.int32),
            pltpu.VMEM((2, _GRP, _CH), jnp.int32),
            pltpu.VMEM((2, _GRP, _CH, _L), jnp.float32),
            pltpu.VMEM_SHARED((_NACC, _L), jnp.float32),
            pltpu.SemaphoreType.DMA((2,)),
            pltpu.SemaphoreType.DMA((2,)),
        ],
        compiler_params=_SC_PARAMS,
    )
    return k(hp, src3, dst3, zeros)


def _sc_head_gather_body(xp_hbm, pos_hbm, g_hbm, pidx, rows, sem):
    c = lax.axis_index("c")
    s = lax.axis_index("s")
    per_w = G_PAD // _NS
    pltpu.sync_copy(pos_hbm.at[pl.ds(s * per_w, per_w)], pidx)
    pltpu.async_copy(xp_hbm.at[c].at[pidx], rows, sem).wait()
    pltpu.sync_copy(rows, g_hbm.at[c, pl.ds(s * per_w, per_w)])


def _sc_head_gather(xp, pos):
    per_w = G_PAD // _NS
    k = pl.kernel(
        _sc_head_gather_body,
        out_type=jax.ShapeDtypeStruct((_NC, G_PAD, _L), jnp.float32),
        mesh=_sc_mesh(),
        scratch_types=[
            pltpu.VMEM((per_w,), jnp.int32),
            pltpu.VMEM((per_w, _L), jnp.float32),
            pltpu.SemaphoreType.DMA,
        ],
        compiler_params=_SC_PARAMS,
    )
    return k(xp, pos)


_RB = 2000


def _pack(h):
    lo = h[:, :16]
    hi = jnp.concatenate([h[:, 16:19], jnp.zeros((h.shape[0], 13), h.dtype)],
                         axis=1)
    return lo, hi


def _k0_body(x_ref, w_ref, b_ref, out_ref):
    h = jnp.dot(x_ref[...], w_ref[...],
                preferred_element_type=jnp.float32) + b_ref[...]
    lo, hi = _pack(h)
    out_ref[0] = lo
    out_ref[1] = hi


def _k0(x, W, b):
    return pl.pallas_call(
        _k0_body,
        grid=(N // _RB,),
        in_specs=[
            pl.BlockSpec((_RB, 11), lambda i: (i, 0)),
            pl.BlockSpec((11, 19), lambda i: (0, 0)),
            pl.BlockSpec((1, 19), lambda i: (0, 0)),
        ],
        out_specs=pl.BlockSpec((2, _RB, 16), lambda i: (0, i, 0)),
        out_shape=jax.ShapeDtypeStruct((2, N, 16), jnp.float32),
    )(x, W, b.reshape(1, 19))


def _mid_body(has_residual, fin, x_ref, agg_ref, w1_ref, b1_ref,
              w0n_ref, b0n_ref, x1_ref, hp_ref):
    a19 = jnp.concatenate([agg_ref[0], agg_ref[1][:, :3]], axis=1)
    cat = jnp.concatenate([x_ref[...], a19], axis=1)
    y = jnp.dot(cat, w1_ref[...], preferred_element_type=jnp.float32) \
        + b1_ref[...]
    if has_residual:
        y = y + x_ref[...][:, :19]
    y = jnp.maximum(y, 0.0)
    x1_ref[...] = y
    h = jnp.dot(y, w0n_ref[...], preferred_element_type=jnp.float32) \
        + b0n_ref[...]
    lo, hi = _pack(h)
    hp_ref[0] = lo
    hp_ref[1] = hi


def _mid(x, agg, W1, b1, W0n, b0n, has_residual):
    fin = x.shape[1]
    body = functools.partial(_mid_body, has_residual, fin)
    return pl.pallas_call(
        body,
        grid=(N // _RB,),
        in_specs=[
            pl.BlockSpec((_RB, fin), lambda i: (i, 0)),
            pl.BlockSpec((2, _RB, 16), lambda i: (0, i, 0)),
            pl.BlockSpec((fin + 19, 19), lambda i: (0, 0)),
            pl.BlockSpec((1, 19), lambda i: (0, 0)),
            pl.BlockSpec((19, 19), lambda i: (0, 0)),
            pl.BlockSpec((1, 19), lambda i: (0, 0)),
        ],
        out_specs=[
            pl.BlockSpec((_RB, 19), lambda i: (i, 0)),
            pl.BlockSpec((2, _RB, 16), lambda i: (0, i, 0)),
        ],
        out_shape=[
            jax.ShapeDtypeStruct((N, 19), jnp.float32),
            jax.ShapeDtypeStruct((2, N, 16), jnp.float32),
        ],
    )(x, agg, W1, b1.reshape(1, 19), W0n, b0n.reshape(1, 19))


def _k3_body(x_ref, agg_ref, w1_ref, b1_ref, out_ref):
    a19 = jnp.concatenate([agg_ref[0], agg_ref[1][:, :3]], axis=1)
    cat = jnp.concatenate([x_ref[...], a19], axis=1)
    y = jnp.dot(cat, w1_ref[...], preferred_element_type=jnp.float32) \
        + b1_ref[...]
    y = jnp.maximum(y + x_ref[...], 0.0)
    lo, hi = _pack(y)
    out_ref[0] = lo
    out_ref[1] = hi


def _k3(x, agg, W1, b1):
    return pl.pallas_call(
        _k3_body,
        grid=(N // _RB,),
        in_specs=[
            pl.BlockSpec((_RB, 19), lambda i: (i, 0)),
            pl.BlockSpec((2, _RB, 16), lambda i: (0, i, 0)),
            pl.BlockSpec((38, 19), lambda i: (0, 0)),
            pl.BlockSpec((1, 19), lambda i: (0, 0)),
        ],
        out_specs=pl.BlockSpec((2, _RB, 16), lambda i: (0, i, 0)),
        out_shape=jax.ShapeDtypeStruct((2, N, 16), jnp.float32),
    )(x, agg, W1, b1.reshape(1, 19))


_PB = 2000


def _pos_body(batch_ref, out_ref):
    i = pl.program_id(0)

    @pl.when(i == 0)
    def _():
        out_ref[...] = jnp.zeros_like(out_ref)

    b = batch_ref[0]
    gids = lax.broadcasted_iota(jnp.int32, (G_PAD, 1), 0)
    le = (b <= gids).astype(jnp.int32)
    out_ref[...] += jnp.sum(le, axis=1, keepdims=True)

    @pl.when(i == (N // _PB) - 1)
    def _():
        out_ref[...] = jnp.maximum(out_ref[...] - 1, 0)


def _pos_kernel(batch):
    out = pl.pallas_call(
        _pos_body,
        grid=(N // _PB,),
        in_specs=[pl.BlockSpec((1, 1, _PB), lambda i: (i, 0, 0))],
        out_specs=pl.BlockSpec((G_PAD, 1), lambda i: (0, 0)),
        out_shape=jax.ShapeDtypeStruct((G_PAD, 1), jnp.int32),
    )(batch.reshape(N // _PB, 1, _PB))
    return out.reshape(G_PAD)


def _head_body(g_ref, w0_ref, b0_ref, w1_ref, b1_ref, out_ref):
    g = jnp.concatenate([g_ref[0, :1000, :], g_ref[1, :1000, :3]], axis=1)
    t = jnp.maximum(
        jnp.dot(g, w0_ref[...], preferred_element_type=jnp.float32)
        + b0_ref[...], 0.0)
    out_ref[...] = jnp.maximum(
        jnp.dot(t, w1_ref[...], preferred_element_type=jnp.float32)
        + b1_ref[...], 0.0)


def _head(g2, Wf0, bf0, Wf1, bf1):
    return pl.pallas_call(
        _head_body,
        in_specs=[
            pl.BlockSpec((2, G_PAD, 16), lambda: (0, 0, 0)),
            pl.BlockSpec((19, 10), lambda: (0, 0)),
            pl.BlockSpec((1, 10), lambda: (0, 0)),
            pl.BlockSpec((10, 1), lambda: (0, 0)),
            pl.BlockSpec((1, 1), lambda: (0, 0)),
        ],
        out_specs=pl.BlockSpec((1000, 1), lambda: (0, 0)),
        out_shape=jax.ShapeDtypeStruct((1000, 1), jnp.float32),
        grid=(),
    )(g2, Wf0, bf0.reshape(1, 10), Wf1, bf1.reshape(1, 1))


def kernel(x, edge_index, batch, W1_0, b1_0, W1_1, b1_1, W2_0, b2_0,
           W2_1, b2_1, W3_0, b3_0, W3_1, b3_1, Wf0, bf0, Wf1, bf1):
    pad = _EP - E
    src3 = jnp.concatenate(
        [edge_index[0], jnp.zeros((pad,), jnp.int32)]).reshape(
            _NS, _NCHUNK, _CH)
    dst3 = jnp.concatenate(
        [edge_index[1], jnp.full((pad,), N, jnp.int32)]).reshape(
            _NS, _NCHUNK, _CH)
    zeros = jnp.zeros((_RPW, _L), jnp.float32)

    hp1 = _k0(x, W1_0, b1_0)
    agg1 = _sc_scatter(hp1, src3, dst3, zeros)
    x1, hp2 = _mid(x, agg1, W1_1, b1_1, W2_0, b2_0, has_residual=False)
    agg2 = _sc_scatter(hp2, src3, dst3, zeros)
    x2, hp3 = _mid(x1, agg2, W2_1, b2_1, W3_0, b3_0, has_residual=True)
    agg3 = _sc_scatter(hp3, src3, dst3, zeros)
    x3p = _k3(x2, agg3, W3_1, b3_1)

    pos = _pos_kernel(batch)
    g2 = _sc_head_gather(x3p, pos)
    return _head(g2, Wf0, bf0, Wf1, bf1)


# speedup vs baseline: 36.0653x; 1.2861x over previous
import functools

import jax
import jax.numpy as jnp
from jax import lax
from jax.experimental import pallas as pl
from jax.experimental.pallas import tpu as pltpu
from jax.experimental.pallas import tpu_sc as plsc

N = 100000
E = 6400000
G_PAD = 1024

_NC, _NS, _L = 2, 16, 16
_CH = 800
_GRP = 1
_EPS = E // _NS
_NIT = _EPS // _CH
_NACC = N
_RPW = 6256
_RPW_LAST = N - (_NS - 1) * _RPW


def _sc_mesh():
    return plsc.VectorSubcoreMesh(
        core_axis_name="c", subcore_axis_name="s",
        num_cores=_NC, num_subcores=_NS)


_SC_PARAMS = pltpu.CompilerParams(use_tc_tiling_on_sc=False)


def _sc_scatter_body(hp_hbm, src_hbm, dst_hbm, zeros_hbm, agg_hbm,
                     idx_s, idx_d, msg, acc, gsem, ssem):
    c = lax.axis_index("c")
    s = lax.axis_index("s")

    @pl.when(s < _NS - 1)
    def _():
        pltpu.sync_copy(zeros_hbm, acc.at[pl.ds(s * _RPW, _RPW)])

    @pl.when(s == _NS - 1)
    def _():
        pltpu.sync_copy(zeros_hbm.at[pl.ds(0, _RPW_LAST)],
                        acc.at[pl.ds((_NS - 1) * _RPW, _RPW_LAST)])

    plsc.subcore_barrier()

    table = hp_hbm.at[c]

    def _load_idx(g, q):
        off = s * _EPS + g * _CH
        pltpu.sync_copy(src_hbm.at[pl.ds(off, _CH)], idx_s.at[q, 0])
        pltpu.sync_copy(dst_hbm.at[pl.ds(off, _CH)], idx_d.at[q, 0])

    def _fire_gathers(q):
        for b in range(_GRP):
            pltpu.async_copy(table.at[idx_s.at[q, b]], msg.at[q, b],
                             gsem.at[q])

    def _wait_gathers(q):
        for b in range(_GRP):
            pltpu.make_async_copy(table.at[idx_s.at[q, b]], msg.at[q, b],
                                  gsem.at[q]).wait()

    def _fire_scatters(q):
        for b in range(_GRP):
            pltpu.async_copy(msg.at[q, b], acc.at[idx_d.at[q, b]],
                             ssem.at[q], add=True)

    def _wait_scatters(q):
        for b in range(_GRP):
            pltpu.make_async_copy(msg.at[q, b], acc.at[idx_d.at[q, b]],
                                  ssem.at[q]).wait()

    _load_idx(0, 0)
    _fire_gathers(0)

    @pl.loop(0, (_NIT - 2) // 2)
    def _(t):
        for q in (0, 1):
            g = t * 2 + q

            @pl.when(g >= 1)
            def _():
                _wait_scatters(1 - q)

            _load_idx(g + 1, 1 - q)
            _fire_gathers(1 - q)
            _wait_gathers(q)
            _fire_scatters(q)

    _wait_scatters(1)
    _load_idx(_NIT - 1, 1)
    _fire_gathers(1)
    _wait_gathers(0)
    _fire_scatters(0)
    _wait_gathers(1)
    _fire_scatters(1)
    _wait_scatters(0)
    _wait_scatters(1)

    plsc.subcore_barrier()

    @pl.when(s < _NS - 1)
    def _():
        pltpu.sync_copy(acc.at[pl.ds(s * _RPW, _RPW)],
                        agg_hbm.at[c, pl.ds(s * _RPW, _RPW)])

    @pl.when(s == _NS - 1)
    def _():
        pltpu.sync_copy(acc.at[pl.ds((_NS - 1) * _RPW, _RPW_LAST)],
                        agg_hbm.at[c, pl.ds((_NS - 1) * _RPW, _RPW_LAST)])


def _sc_scatter(hp, src3, dst3, zeros):
    k = pl.kernel(
        _sc_scatter_body,
        out_type=jax.ShapeDtypeStruct((_NC, N, _L), jnp.float32),
        mesh=_sc_mesh(),
        scratch_types=[
            pltpu.VMEM((2, _GRP, _CH), jnp.int32),
            pltpu.VMEM((2, _GRP, _CH), jnp.int32),
            pltpu.VMEM((2, _GRP, _CH, _L), jnp.float32),
            pltpu.VMEM_SHARED((_NACC, _L), jnp.float32),
            pltpu.SemaphoreType.DMA((2,)),
            pltpu.SemaphoreType.DMA((2,)),
        ],
        compiler_params=_SC_PARAMS,
    )
    return k(hp, src3, dst3, zeros)


def _sc_head_gather_body(xp_hbm, pos_hbm, g_hbm, pidx, rows, sem):
    c = lax.axis_index("c")
    s = lax.axis_index("s")
    per_w = G_PAD // _NS
    pltpu.sync_copy(pos_hbm.at[pl.ds(s * per_w, per_w)], pidx)
    pltpu.async_copy(xp_hbm.at[c].at[pidx], rows, sem).wait()
    pltpu.sync_copy(rows, g_hbm.at[c, pl.ds(s * per_w, per_w)])


def _sc_head_gather(xp, pos):
    per_w = G_PAD // _NS
    k = pl.kernel(
        _sc_head_gather_body,
        out_type=jax.ShapeDtypeStruct((_NC, G_PAD, _L), jnp.float32),
        mesh=_sc_mesh(),
        scratch_types=[
            pltpu.VMEM((per_w,), jnp.int32),
            pltpu.VMEM((per_w, _L), jnp.float32),
            pltpu.SemaphoreType.DMA,
        ],
        compiler_params=_SC_PARAMS,
    )
    return k(xp, pos)


_RB = 2000


def _pack(h):
    lo = h[:, :16]
    hi = jnp.concatenate([h[:, 16:19], jnp.zeros((h.shape[0], 13), h.dtype)],
                         axis=1)
    return lo, hi


def _k0_body(x_ref, w_ref, b_ref, out_ref):
    h = jnp.dot(x_ref[...], w_ref[...],
                preferred_element_type=jnp.float32) + b_ref[...]
    lo, hi = _pack(h)
    out_ref[0] = lo
    out_ref[1] = hi


def _k0(x, W, b):
    return pl.pallas_call(
        _k0_body,
        grid=(N // _RB,),
        in_specs=[
            pl.BlockSpec((_RB, 11), lambda i: (i, 0)),
            pl.BlockSpec((11, 19), lambda i: (0, 0)),
            pl.BlockSpec((1, 19), lambda i: (0, 0)),
        ],
        out_specs=pl.BlockSpec((2, _RB, 16), lambda i: (0, i, 0)),
        out_shape=jax.ShapeDtypeStruct((2, N, 16), jnp.float32),
    )(x, W, b.reshape(1, 19))


def _mid_body(has_residual, fin, x_ref, agg_ref, w1_ref, b1_ref,
              w0n_ref, b0n_ref, x1_ref, hp_ref):
    a19 = jnp.concatenate([agg_ref[0], agg_ref[1][:, :3]], axis=1)
    cat = jnp.concatenate([x_ref[...], a19], axis=1)
    y = jnp.dot(cat, w1_ref[...], preferred_element_type=jnp.float32) \
        + b1_ref[...]
    if has_residual:
        y = y + x_ref[...][:, :19]
    y = jnp.maximum(y, 0.0)
    x1_ref[...] = y
    h = jnp.dot(y, w0n_ref[...], preferred_element_type=jnp.float32) \
        + b0n_ref[...]
    lo, hi = _pack(h)
    hp_ref[0] = lo
    hp_ref[1] = hi


def _mid(x, agg, W1, b1, W0n, b0n, has_residual):
    fin = x.shape[1]
    body = functools.partial(_mid_body, has_residual, fin)
    return pl.pallas_call(
        body,
        grid=(N // _RB,),
        in_specs=[
            pl.BlockSpec((_RB, fin), lambda i: (i, 0)),
            pl.BlockSpec((2, _RB, 16), lambda i: (0, i, 0)),
            pl.BlockSpec((fin + 19, 19), lambda i: (0, 0)),
            pl.BlockSpec((1, 19), lambda i: (0, 0)),
            pl.BlockSpec((19, 19), lambda i: (0, 0)),
            pl.BlockSpec((1, 19), lambda i: (0, 0)),
        ],
        out_specs=[
            pl.BlockSpec((_RB, 19), lambda i: (i, 0)),
            pl.BlockSpec((2, _RB, 16), lambda i: (0, i, 0)),
        ],
        out_shape=[
            jax.ShapeDtypeStruct((N, 19), jnp.float32),
            jax.ShapeDtypeStruct((2, N, 16), jnp.float32),
        ],
    )(x, agg, W1, b1.reshape(1, 19), W0n, b0n.reshape(1, 19))


def _k3_body(x_ref, agg_ref, w1_ref, b1_ref, out_ref):
    a19 = jnp.concatenate([agg_ref[0], agg_ref[1][:, :3]], axis=1)
    cat = jnp.concatenate([x_ref[...], a19], axis=1)
    y = jnp.dot(cat, w1_ref[...], preferred_element_type=jnp.float32) \
        + b1_ref[...]
    y = jnp.maximum(y + x_ref[...], 0.0)
    lo, hi = _pack(y)
    out_ref[0] = lo
    out_ref[1] = hi


def _k3(x, agg, W1, b1):
    return pl.pallas_call(
        _k3_body,
        grid=(N // _RB,),
        in_specs=[
            pl.BlockSpec((_RB, 19), lambda i: (i, 0)),
            pl.BlockSpec((2, _RB, 16), lambda i: (0, i, 0)),
            pl.BlockSpec((38, 19), lambda i: (0, 0)),
            pl.BlockSpec((1, 19), lambda i: (0, 0)),
        ],
        out_specs=pl.BlockSpec((2, _RB, 16), lambda i: (0, i, 0)),
        out_shape=jax.ShapeDtypeStruct((2, N, 16), jnp.float32),
    )(x, agg, W1, b1.reshape(1, 19))


_PB = 2000


def _pos_body(batch_ref, out_ref):
    i = pl.program_id(0)

    @pl.when(i == 0)
    def _():
        out_ref[...] = jnp.zeros_like(out_ref)

    b = batch_ref[0]
    gids = lax.broadcasted_iota(jnp.int32, (G_PAD, 1), 0)
    le = (b <= gids).astype(jnp.int32)
    out_ref[...] += jnp.sum(le, axis=1, keepdims=True)

    @pl.when(i == (N // _PB) - 1)
    def _():
        out_ref[...] = jnp.maximum(out_ref[...] - 1, 0)


def _pos_kernel(batch):
    out = pl.pallas_call(
        _pos_body,
        grid=(N // _PB,),
        in_specs=[pl.BlockSpec((1, 1, _PB), lambda i: (i, 0, 0))],
        out_specs=pl.BlockSpec((G_PAD, 1), lambda i: (0, 0)),
        out_shape=jax.ShapeDtypeStruct((G_PAD, 1), jnp.int32),
    )(batch.reshape(N // _PB, 1, _PB))
    return out.reshape(G_PAD)


def _head_body(g_ref, w0_ref, b0_ref, w1_ref, b1_ref, out_ref):
    g = jnp.concatenate([g_ref[0, :1000, :], g_ref[1, :1000, :3]], axis=1)
    t = jnp.maximum(
        jnp.dot(g, w0_ref[...], preferred_element_type=jnp.float32)
        + b0_ref[...], 0.0)
    out_ref[...] = jnp.maximum(
        jnp.dot(t, w1_ref[...], preferred_element_type=jnp.float32)
        + b1_ref[...], 0.0)


def _head(g2, Wf0, bf0, Wf1, bf1):
    return pl.pallas_call(
        _head_body,
        in_specs=[
            pl.BlockSpec((2, G_PAD, 16), lambda: (0, 0, 0)),
            pl.BlockSpec((19, 10), lambda: (0, 0)),
            pl.BlockSpec((1, 10), lambda: (0, 0)),
            pl.BlockSpec((10, 1), lambda: (0, 0)),
            pl.BlockSpec((1, 1), lambda: (0, 0)),
        ],
        out_specs=pl.BlockSpec((1000, 1), lambda: (0, 0)),
        out_shape=jax.ShapeDtypeStruct((1000, 1), jnp.float32),
        grid=(),
    )(g2, Wf0, bf0.reshape(1, 10), Wf1, bf1.reshape(1, 1))


def kernel(x, edge_index, batch, W1_0, b1_0, W1_1, b1_1, W2_0, b2_0,
           W2_1, b2_1, W3_0, b3_0, W3_1, b3_1, Wf0, bf0, Wf1, bf1):
    src3 = edge_index[0]
    dst3 = edge_index[1]
    zeros = jnp.zeros((_RPW, _L), jnp.float32)

    hp1 = _k0(x, W1_0, b1_0)
    agg1 = _sc_scatter(hp1, src3, dst3, zeros)
    x1, hp2 = _mid(x, agg1, W1_1, b1_1, W2_0, b2_0, has_residual=False)
    agg2 = _sc_scatter(hp2, src3, dst3, zeros)
    x2, hp3 = _mid(x1, agg2, W2_1, b2_1, W3_0, b3_0, has_residual=True)
    agg3 = _sc_scatter(hp3, src3, dst3, zeros)
    x3p = _k3(x2, agg3, W3_1, b3_1)

    pos = _pos_kernel(batch)
    g2 = _sc_head_gather(x3p, pos)
    return _head(g2, Wf0, bf0, Wf1, bf1)


# speedup vs baseline: 40.6156x; 1.1262x over previous
import functools

import jax
import jax.numpy as jnp
from jax import lax
from jax.experimental import pallas as pl
from jax.experimental.pallas import tpu as pltpu
from jax.experimental.pallas import tpu_sc as plsc

N = 100000
E = 6400000
G_PAD = 1024

_NC, _NS, _L = 2, 16, 16
_CH = 400
_GRP = 1
_EPS = E // _NS
_NIT = _EPS // _CH
_NACC = N
_RPW = 6256
_RPW_LAST = N - (_NS - 1) * _RPW


def _sc_mesh():
    return plsc.VectorSubcoreMesh(
        core_axis_name="c", subcore_axis_name="s",
        num_cores=_NC, num_subcores=_NS)


_SC_PARAMS = pltpu.CompilerParams(use_tc_tiling_on_sc=False)


def _sc_scatter_body(hp_hbm, src_hbm, dst_hbm, zeros_hbm, agg_hbm,
                     idx_s, idx_d, msg, acc, gsem, ssem, isem):
    c = lax.axis_index("c")
    s = lax.axis_index("s")

    @pl.when(s < _NS - 1)
    def _():
        pltpu.sync_copy(zeros_hbm, acc.at[pl.ds(s * _RPW, _RPW)])

    @pl.when(s == _NS - 1)
    def _():
        pltpu.sync_copy(zeros_hbm.at[pl.ds(0, _RPW_LAST)],
                        acc.at[pl.ds((_NS - 1) * _RPW, _RPW_LAST)])

    plsc.subcore_barrier()

    table = hp_hbm.at[c]

    def _load_idx_async(g, r):
        off = s * _EPS + g * _CH
        pltpu.async_copy(src_hbm.at[pl.ds(off, _CH)], idx_s.at[r, 0],
                         isem.at[r])
        pltpu.async_copy(dst_hbm.at[pl.ds(off, _CH)], idx_d.at[r, 0],
                         isem.at[r])

    def _wait_idx(g, r):
        off = s * _EPS + g * _CH
        pltpu.make_async_copy(src_hbm.at[pl.ds(off, _CH)], idx_s.at[r, 0],
                              isem.at[r]).wait()
        pltpu.make_async_copy(dst_hbm.at[pl.ds(off, _CH)], idx_d.at[r, 0],
                              isem.at[r]).wait()

    def _fire_gathers(q, r):
        pltpu.async_copy(table.at[idx_s.at[r, 0]], msg.at[q, 0], gsem.at[q])

    def _wait_gathers(q, r):
        pltpu.make_async_copy(table.at[idx_s.at[r, 0]], msg.at[q, 0],
                              gsem.at[q]).wait()

    def _fire_scatters(q, r):
        pltpu.async_copy(msg.at[q, 0], acc.at[idx_d.at[r, 0]],
                         ssem.at[q], add=True)

    def _wait_scatters(q, r):
        pltpu.make_async_copy(msg.at[q, 0], acc.at[idx_d.at[r, 0]],
                              ssem.at[q]).wait()

    def _step(g, qq, fire_next=True):
        q = qq % 2
        r = qq % 4

        @pl.when(g >= 1)
        def _():
            _wait_scatters(1 - q, (qq - 1) % 4)

        @pl.when(g + 3 < _NIT)
        def _():
            _load_idx_async(g + 3, (qq + 3) % 4)

        if fire_next:
            _wait_idx(g + 1, (qq + 1) % 4)
            _fire_gathers(1 - q, (qq + 1) % 4)
        _wait_gathers(q, r)
        _fire_scatters(q, r)

    _load_idx_async(0, 0)
    _load_idx_async(1, 1)
    _load_idx_async(2, 2)
    _wait_idx(0, 0)
    _fire_gathers(0, 0)

    @pl.loop(0, (_NIT - 4) // 4)
    def _(t):
        for qq in range(4):
            _step(t * 4 + qq, qq)

    _step(_NIT - 4, 0)
    _step(_NIT - 3, 1)
    _step(_NIT - 2, 2)
    _step(_NIT - 1, 3, fire_next=False)
    _wait_scatters(1, 3)

    plsc.subcore_barrier()

    @pl.when(s < _NS - 1)
    def _():
        pltpu.sync_copy(acc.at[pl.ds(s * _RPW, _RPW)],
                        agg_hbm.at[c, pl.ds(s * _RPW, _RPW)])

    @pl.when(s == _NS - 1)
    def _():
        pltpu.sync_copy(acc.at[pl.ds((_NS - 1) * _RPW, _RPW_LAST)],
                        agg_hbm.at[c, pl.ds((_NS - 1) * _RPW, _RPW_LAST)])


def _sc_scatter(hp, src3, dst3, zeros):
    k = pl.kernel(
        _sc_scatter_body,
        out_type=jax.ShapeDtypeStruct((_NC, N, _L), jnp.float32),
        mesh=_sc_mesh(),
        scratch_types=[
            pltpu.VMEM((4, 1, _CH), jnp.int32),
            pltpu.VMEM((4, 1, _CH), jnp.int32),
            pltpu.VMEM((2, 1, _CH, _L), jnp.float32),
            pltpu.VMEM_SHARED((_NACC, _L), jnp.float32),
            pltpu.SemaphoreType.DMA((2,)),
            pltpu.SemaphoreType.DMA((2,)),
            pltpu.SemaphoreType.DMA((4,)),
        ],
        compiler_params=_SC_PARAMS,
    )
    return k(hp, src3, dst3, zeros)


def _sc_head_gather_body(xp_hbm, pos_hbm, g_hbm, pidx, rows, sem):
    c = lax.axis_index("c")
    s = lax.axis_index("s")
    per_w = G_PAD // _NS
    pltpu.sync_copy(pos_hbm.at[pl.ds(s * per_w, per_w)], pidx)
    pltpu.async_copy(xp_hbm.at[c].at[pidx], rows, sem).wait()
    pltpu.sync_copy(rows, g_hbm.at[c, pl.ds(s * per_w, per_w)])


def _sc_head_gather(xp, pos):
    per_w = G_PAD // _NS
    k = pl.kernel(
        _sc_head_gather_body,
        out_type=jax.ShapeDtypeStruct((_NC, G_PAD, _L), jnp.float32),
        mesh=_sc_mesh(),
        scratch_types=[
            pltpu.VMEM((per_w,), jnp.int32),
            pltpu.VMEM((per_w, _L), jnp.float32),
            pltpu.SemaphoreType.DMA,
        ],
        compiler_params=_SC_PARAMS,
    )
    return k(xp, pos)


_RB = 2000


def _pack(h):
    lo = h[:, :16]
    hi = jnp.concatenate([h[:, 16:19], jnp.zeros((h.shape[0], 13), h.dtype)],
                         axis=1)
    return lo, hi


def _k0_body(x_ref, w_ref, b_ref, out_ref):
    h = jnp.dot(x_ref[...], w_ref[...],
                preferred_element_type=jnp.float32) + b_ref[...]
    lo, hi = _pack(h)
    out_ref[0] = lo
    out_ref[1] = hi


def _k0(x, W, b):
    return pl.pallas_call(
        _k0_body,
        grid=(N // _RB,),
        in_specs=[
            pl.BlockSpec((_RB, 11), lambda i: (i, 0)),
            pl.BlockSpec((11, 19), lambda i: (0, 0)),
            pl.BlockSpec((1, 19), lambda i: (0, 0)),
        ],
        out_specs=pl.BlockSpec((2, _RB, 16), lambda i: (0, i, 0)),
        out_shape=jax.ShapeDtypeStruct((2, N, 16), jnp.float32),
    )(x, W, b.reshape(1, 19))


def _mid_body(has_residual, fin, x_ref, agg_ref, w1_ref, b1_ref,
              w0n_ref, b0n_ref, x1_ref, hp_ref):
    a19 = jnp.concatenate([agg_ref[0], agg_ref[1][:, :3]], axis=1)
    cat = jnp.concatenate([x_ref[...], a19], axis=1)
    y = jnp.dot(cat, w1_ref[...], preferred_element_type=jnp.float32) \
        + b1_ref[...]
    if has_residual:
        y = y + x_ref[...][:, :19]
    y = jnp.maximum(y, 0.0)
    x1_ref[...] = y
    h = jnp.dot(y, w0n_ref[...], preferred_element_type=jnp.float32) \
        + b0n_ref[...]
    lo, hi = _pack(h)
    hp_ref[0] = lo
    hp_ref[1] = hi


def _mid(x, agg, W1, b1, W0n, b0n, has_residual):
    fin = x.shape[1]
    body = functools.partial(_mid_body, has_residual, fin)
    return pl.pallas_call(
        body,
        grid=(N // _RB,),
        in_specs=[
            pl.BlockSpec((_RB, fin), lambda i: (i, 0)),
            pl.BlockSpec((2, _RB, 16), lambda i: (0, i, 0)),
            pl.BlockSpec((fin + 19, 19), lambda i: (0, 0)),
            pl.BlockSpec((1, 19), lambda i: (0, 0)),
            pl.BlockSpec((19, 19), lambda i: (0, 0)),
            pl.BlockSpec((1, 19), lambda i: (0, 0)),
        ],
        out_specs=[
            pl.BlockSpec((_RB, 19), lambda i: (i, 0)),
            pl.BlockSpec((2, _RB, 16), lambda i: (0, i, 0)),
        ],
        out_shape=[
            jax.ShapeDtypeStruct((N, 19), jnp.float32),
            jax.ShapeDtypeStruct((2, N, 16), jnp.float32),
        ],
    )(x, agg, W1, b1.reshape(1, 19), W0n, b0n.reshape(1, 19))


def _k3_body(x_ref, agg_ref, w1_ref, b1_ref, out_ref):
    a19 = jnp.concatenate([agg_ref[0], agg_ref[1][:, :3]], axis=1)
    cat = jnp.concatenate([x_ref[...], a19], axis=1)
    y = jnp.dot(cat, w1_ref[...], preferred_element_type=jnp.float32) \
        + b1_ref[...]
    y = jnp.maximum(y + x_ref[...], 0.0)
    lo, hi = _pack(y)
    out_ref[0] = lo
    out_ref[1] = hi


def _k3(x, agg, W1, b1):
    return pl.pallas_call(
        _k3_body,
        grid=(N // _RB,),
        in_specs=[
            pl.BlockSpec((_RB, 19), lambda i: (i, 0)),
            pl.BlockSpec((2, _RB, 16), lambda i: (0, i, 0)),
            pl.BlockSpec((38, 19), lambda i: (0, 0)),
            pl.BlockSpec((1, 19), lambda i: (0, 0)),
        ],
        out_specs=pl.BlockSpec((2, _RB, 16), lambda i: (0, i, 0)),
        out_shape=jax.ShapeDtypeStruct((2, N, 16), jnp.float32),
    )(x, agg, W1, b1.reshape(1, 19))


_PB = 2000


def _pos_body(batch_ref, out_ref):
    i = pl.program_id(0)

    @pl.when(i == 0)
    def _():
        out_ref[...] = jnp.zeros_like(out_ref)

    b = batch_ref[0]
    gids = lax.broadcasted_iota(jnp.int32, (G_PAD, 1), 0)
    le = (b <= gids).astype(jnp.int32)
    out_ref[...] += jnp.sum(le, axis=1, keepdims=True)

    @pl.when(i == (N // _PB) - 1)
    def _():
        out_ref[...] = jnp.maximum(out_ref[...] - 1, 0)


def _pos_kernel(batch):
    out = pl.pallas_call(
        _pos_body,
        grid=(N // _PB,),
        in_specs=[pl.BlockSpec((1, 1, _PB), lambda i: (i, 0, 0))],
        out_specs=pl.BlockSpec((G_PAD, 1), lambda i: (0, 0)),
        out_shape=jax.ShapeDtypeStruct((G_PAD, 1), jnp.int32),
    )(batch.reshape(N // _PB, 1, _PB))
    return out.reshape(G_PAD)


def _head_body(g_ref, w0_ref, b0_ref, w1_ref, b1_ref, out_ref):
    g = jnp.concatenate([g_ref[0, :1000, :], g_ref[1, :1000, :3]], axis=1)
    t = jnp.maximum(
        jnp.dot(g, w0_ref[...], preferred_element_type=jnp.float32)
        + b0_ref[...], 0.0)
    out_ref[...] = jnp.maximum(
        jnp.dot(t, w1_ref[...], preferred_element_type=jnp.float32)
        + b1_ref[...], 0.0)


def _head(g2, Wf0, bf0, Wf1, bf1):
    return pl.pallas_call(
        _head_body,
        in_specs=[
            pl.BlockSpec((2, G_PAD, 16), lambda: (0, 0, 0)),
            pl.BlockSpec((19, 10), lambda: (0, 0)),
            pl.BlockSpec((1, 10), lambda: (0, 0)),
            pl.BlockSpec((10, 1), lambda: (0, 0)),
            pl.BlockSpec((1, 1), lambda: (0, 0)),
        ],
        out_specs=pl.BlockSpec((1000, 1), lambda: (0, 0)),
        out_shape=jax.ShapeDtypeStruct((1000, 1), jnp.float32),
        grid=(),
    )(g2, Wf0, bf0.reshape(1, 10), Wf1, bf1.reshape(1, 1))


def kernel(x, edge_index, batch, W1_0, b1_0, W1_1, b1_1, W2_0, b2_0,
           W2_1, b2_1, W3_0, b3_0, W3_1, b3_1, Wf0, bf0, Wf1, bf1):
    src3 = edge_index[0]
    dst3 = edge_index[1]
    zeros = jnp.zeros((_RPW, _L), jnp.float32)

    hp1 = _k0(x, W1_0, b1_0)
    agg1 = _sc_scatter(hp1, src3, dst3, zeros)
    x1, hp2 = _mid(x, agg1, W1_1, b1_1, W2_0, b2_0, has_residual=False)
    agg2 = _sc_scatter(hp2, src3, dst3, zeros)
    x2, hp3 = _mid(x1, agg2, W2_1, b2_1, W3_0, b3_0, has_residual=True)
    agg3 = _sc_scatter(hp3, src3, dst3, zeros)
    x3p = _k3(x2, agg3, W3_1, b3_1)

    pos = _pos_kernel(batch)
    g2 = _sc_head_gather(x3p, pos)
    return _head(g2, Wf0, bf0, Wf1, bf1)


# speedup vs baseline: 47.6124x; 1.1723x over previous
import functools

import jax
import jax.numpy as jnp
from jax import lax
from jax.experimental import pallas as pl
from jax.experimental.pallas import tpu as pltpu
from jax.experimental.pallas import tpu_sc as plsc

N = 100000
E = 6400000
G_PAD = 1024

_NC, _NS, _L = 2, 16, 16
_CH = 200
_GRP = 1
_EPS = E // _NS
_NIT = _EPS // _CH
_NACC = N
_RPW = 6256
_RPW_LAST = N - (_NS - 1) * _RPW


def _sc_mesh():
    return plsc.VectorSubcoreMesh(
        core_axis_name="c", subcore_axis_name="s",
        num_cores=_NC, num_subcores=_NS)


_SC_PARAMS = pltpu.CompilerParams(use_tc_tiling_on_sc=False)


def _sc_scatter_body(hp_hbm, src_hbm, dst_hbm, zeros_hbm, agg_hbm,
                     idx_s, idx_d, msg, acc, gsem, ssem, isem):
    c = lax.axis_index("c")
    s = lax.axis_index("s")

    @pl.when(s < _NS - 1)
    def _():
        pltpu.sync_copy(zeros_hbm, acc.at[pl.ds(s * _RPW, _RPW)])

    @pl.when(s == _NS - 1)
    def _():
        pltpu.sync_copy(zeros_hbm.at[pl.ds(0, _RPW_LAST)],
                        acc.at[pl.ds((_NS - 1) * _RPW, _RPW_LAST)])

    plsc.subcore_barrier()

    table = hp_hbm.at[c]

    def _load_idx_async(g, r):
        off = s * _EPS + g * _CH
        pltpu.async_copy(src_hbm.at[pl.ds(off, _CH)], idx_s.at[r, 0],
                         isem.at[r])
        pltpu.async_copy(dst_hbm.at[pl.ds(off, _CH)], idx_d.at[r, 0],
                         isem.at[r])

    def _wait_idx(g, r):
        off = s * _EPS + g * _CH
        pltpu.make_async_copy(src_hbm.at[pl.ds(off, _CH)], idx_s.at[r, 0],
                              isem.at[r]).wait()
        pltpu.make_async_copy(dst_hbm.at[pl.ds(off, _CH)], idx_d.at[r, 0],
                              isem.at[r]).wait()

    def _fire_gathers(q, r):
        pltpu.async_copy(table.at[idx_s.at[r, 0]], msg.at[q, 0], gsem.at[q])

    def _wait_gathers(q, r):
        pltpu.make_async_copy(table.at[idx_s.at[r, 0]], msg.at[q, 0],
                              gsem.at[q]).wait()

    def _fire_scatters(q, r):
        pltpu.async_copy(msg.at[q, 0], acc.at[idx_d.at[r, 0]],
                         ssem.at[q], add=True)

    def _wait_scatters(q, r):
        pltpu.make_async_copy(msg.at[q, 0], acc.at[idx_d.at[r, 0]],
                              ssem.at[q]).wait()

    def _step(g, qq):
        p = qq % 4

        @pl.when(g >= 1)
        def _():
            _wait_scatters((qq - 1) % 4, (qq - 1) % 8)

        @pl.when(g + 7 < _NIT)
        def _():
            _load_idx_async(g + 7, (qq + 7) % 8)

        @pl.when(g + 3 < _NIT)
        def _():
            _wait_idx(g + 3, (qq + 3) % 8)
            _fire_gathers((qq + 3) % 4, (qq + 3) % 8)

        _wait_gathers(p, qq % 8)
        _fire_scatters(p, qq % 8)

    for j in range(7):
        _load_idx_async(j, j)
    for g0 in range(3):
        _wait_idx(g0, g0)
        _fire_gathers(g0 % 4, g0)

    @pl.loop(0, (_NIT - 8) // 8)
    def _(t):
        for qq in range(8):
            _step(t * 8 + qq, qq)

    for qq in range(8):
        _step(_NIT - 8 + qq, qq)
    _wait_scatters(3, 7)

    plsc.subcore_barrier()

    @pl.when(s < _NS - 1)
    def _():
        pltpu.sync_copy(acc.at[pl.ds(s * _RPW, _RPW)],
                        agg_hbm.at[c, pl.ds(s * _RPW, _RPW)])

    @pl.when(s == _NS - 1)
    def _():
        pltpu.sync_copy(acc.at[pl.ds((_NS - 1) * _RPW, _RPW_LAST)],
                        agg_hbm.at[c, pl.ds((_NS - 1) * _RPW, _RPW_LAST)])


def _sc_scatter(hp, src3, dst3, zeros):
    k = pl.kernel(
        _sc_scatter_body,
        out_type=jax.ShapeDtypeStruct((_NC, N, _L), jnp.float32),
        mesh=_sc_mesh(),
        scratch_types=[
            pltpu.VMEM((8, 1, _CH), jnp.int32),
            pltpu.VMEM((8, 1, _CH), jnp.int32),
            pltpu.VMEM((4, 1, _CH, _L), jnp.float32),
            pltpu.VMEM_SHARED((_NACC, _L), jnp.float32),
            pltpu.SemaphoreType.DMA((4,)),
            pltpu.SemaphoreType.DMA((4,)),
            pltpu.SemaphoreType.DMA((8,)),
        ],
        compiler_params=_SC_PARAMS,
    )
    return k(hp, src3, dst3, zeros)


def _sc_head_gather_body(xp_hbm, pos_hbm, g_hbm, pidx, rows, sem):
    c = lax.axis_index("c")
    s = lax.axis_index("s")
    per_w = G_PAD // _NS
    pltpu.sync_copy(pos_hbm.at[pl.ds(s * per_w, per_w)], pidx)
    pltpu.async_copy(xp_hbm.at[c].at[pidx], rows, sem).wait()
    pltpu.sync_copy(rows, g_hbm.at[c, pl.ds(s * per_w, per_w)])


def _sc_head_gather(xp, pos):
    per_w = G_PAD // _NS
    k = pl.kernel(
        _sc_head_gather_body,
        out_type=jax.ShapeDtypeStruct((_NC, G_PAD, _L), jnp.float32),
        mesh=_sc_mesh(),
        scratch_types=[
            pltpu.VMEM((per_w,), jnp.int32),
            pltpu.VMEM((per_w, _L), jnp.float32),
            pltpu.SemaphoreType.DMA,
        ],
        compiler_params=_SC_PARAMS,
    )
    return k(xp, pos)


_RB = 2000


def _pack(h):
    lo = h[:, :16]
    hi = jnp.concatenate([h[:, 16:19], jnp.zeros((h.shape[0], 13), h.dtype)],
                         axis=1)
    return lo, hi


def _k0_body(x_ref, w_ref, b_ref, out_ref):
    h = jnp.dot(x_ref[...], w_ref[...],
                preferred_element_type=jnp.float32) + b_ref[...]
    lo, hi = _pack(h)
    out_ref[0] = lo
    out_ref[1] = hi


def _k0(x, W, b):
    return pl.pallas_call(
        _k0_body,
        grid=(N // _RB,),
        in_specs=[
            pl.BlockSpec((_RB, 11), lambda i: (i, 0)),
            pl.BlockSpec((11, 19), lambda i: (0, 0)),
            pl.BlockSpec((1, 19), lambda i: (0, 0)),
        ],
        out_specs=pl.BlockSpec((2, _RB, 16), lambda i: (0, i, 0)),
        out_shape=jax.ShapeDtypeStruct((2, N, 16), jnp.float32),
    )(x, W, b.reshape(1, 19))


def _mid_body(has_residual, fin, x_ref, agg_ref, w1_ref, b1_ref,
              w0n_ref, b0n_ref, x1_ref, hp_ref):
    a19 = jnp.concatenate([agg_ref[0], agg_ref[1][:, :3]], axis=1)
    cat = jnp.concatenate([x_ref[...], a19], axis=1)
    y = jnp.dot(cat, w1_ref[...], preferred_element_type=jnp.float32) \
        + b1_ref[...]
    if has_residual:
        y = y + x_ref[...][:, :19]
    y = jnp.maximum(y, 0.0)
    x1_ref[...] = y
    h = jnp.dot(y, w0n_ref[...], preferred_element_type=jnp.float32) \
        + b0n_ref[...]
    lo, hi = _pack(h)
    hp_ref[0] = lo
    hp_ref[1] = hi


def _mid(x, agg, W1, b1, W0n, b0n, has_residual):
    fin = x.shape[1]
    body = functools.partial(_mid_body, has_residual, fin)
    return pl.pallas_call(
        body,
        grid=(N // _RB,),
        in_specs=[
            pl.BlockSpec((_RB, fin), lambda i: (i, 0)),
            pl.BlockSpec((2, _RB, 16), lambda i: (0, i, 0)),
            pl.BlockSpec((fin + 19, 19), lambda i: (0, 0)),
            pl.BlockSpec((1, 19), lambda i: (0, 0)),
            pl.BlockSpec((19, 19), lambda i: (0, 0)),
            pl.BlockSpec((1, 19), lambda i: (0, 0)),
        ],
        out_specs=[
            pl.BlockSpec((_RB, 19), lambda i: (i, 0)),
            pl.BlockSpec((2, _RB, 16), lambda i: (0, i, 0)),
        ],
        out_shape=[
            jax.ShapeDtypeStruct((N, 19), jnp.float32),
            jax.ShapeDtypeStruct((2, N, 16), jnp.float32),
        ],
    )(x, agg, W1, b1.reshape(1, 19), W0n, b0n.reshape(1, 19))


def _k3_body(x_ref, agg_ref, w1_ref, b1_ref, out_ref):
    a19 = jnp.concatenate([agg_ref[0], agg_ref[1][:, :3]], axis=1)
    cat = jnp.concatenate([x_ref[...], a19], axis=1)
    y = jnp.dot(cat, w1_ref[...], preferred_element_type=jnp.float32) \
        + b1_ref[...]
    y = jnp.maximum(y + x_ref[...], 0.0)
    lo, hi = _pack(y)
    out_ref[0] = lo
    out_ref[1] = hi


def _k3(x, agg, W1, b1):
    return pl.pallas_call(
        _k3_body,
        grid=(N // _RB,),
        in_specs=[
            pl.BlockSpec((_RB, 19), lambda i: (i, 0)),
            pl.BlockSpec((2, _RB, 16), lambda i: (0, i, 0)),
            pl.BlockSpec((38, 19), lambda i: (0, 0)),
            pl.BlockSpec((1, 19), lambda i: (0, 0)),
        ],
        out_specs=pl.BlockSpec((2, _RB, 16), lambda i: (0, i, 0)),
        out_shape=jax.ShapeDtypeStruct((2, N, 16), jnp.float32),
    )(x, agg, W1, b1.reshape(1, 19))


_PB = 2000


def _pos_body(batch_ref, out_ref):
    i = pl.program_id(0)

    @pl.when(i == 0)
    def _():
        out_ref[...] = jnp.zeros_like(out_ref)

    b = batch_ref[0]
    gids = lax.broadcasted_iota(jnp.int32, (G_PAD, 1), 0)
    le = (b <= gids).astype(jnp.int32)
    out_ref[...] += jnp.sum(le, axis=1, keepdims=True)

    @pl.when(i == (N // _PB) - 1)
    def _():
        out_ref[...] = jnp.maximum(out_ref[...] - 1, 0)


def _pos_kernel(batch):
    out = pl.pallas_call(
        _pos_body,
        grid=(N // _PB,),
        in_specs=[pl.BlockSpec((1, 1, _PB), lambda i: (i, 0, 0))],
        out_specs=pl.BlockSpec((G_PAD, 1), lambda i: (0, 0)),
        out_shape=jax.ShapeDtypeStruct((G_PAD, 1), jnp.int32),
    )(batch.reshape(N // _PB, 1, _PB))
    return out.reshape(G_PAD)


def _head_body(g_ref, w0_ref, b0_ref, w1_ref, b1_ref, out_ref):
    g = jnp.concatenate([g_ref[0, :1000, :], g_ref[1, :1000, :3]], axis=1)
    t = jnp.maximum(
        jnp.dot(g, w0_ref[...], preferred_element_type=jnp.float32)
        + b0_ref[...], 0.0)
    out_ref[...] = jnp.maximum(
        jnp.dot(t, w1_ref[...], preferred_element_type=jnp.float32)
        + b1_ref[...], 0.0)


def _head(g2, Wf0, bf0, Wf1, bf1):
    return pl.pallas_call(
        _head_body,
        in_specs=[
            pl.BlockSpec((2, G_PAD, 16), lambda: (0, 0, 0)),
            pl.BlockSpec((19, 10), lambda: (0, 0)),
            pl.BlockSpec((1, 10), lambda: (0, 0)),
            pl.BlockSpec((10, 1), lambda: (0, 0)),
            pl.BlockSpec((1, 1), lambda: (0, 0)),
        ],
        out_specs=pl.BlockSpec((1000, 1), lambda: (0, 0)),
        out_shape=jax.ShapeDtypeStruct((1000, 1), jnp.float32),
        grid=(),
    )(g2, Wf0, bf0.reshape(1, 10), Wf1, bf1.reshape(1, 1))


def kernel(x, edge_index, batch, W1_0, b1_0, W1_1, b1_1, W2_0, b2_0,
           W2_1, b2_1, W3_0, b3_0, W3_1, b3_1, Wf0, bf0, Wf1, bf1):
    src3 = edge_index[0]
    dst3 = edge_index[1]
    zeros = jnp.zeros((_RPW, _L), jnp.float32)

    hp1 = _k0(x, W1_0, b1_0)
    agg1 = _sc_scatter(hp1, src3, dst3, zeros)
    x1, hp2 = _mid(x, agg1, W1_1, b1_1, W2_0, b2_0, has_residual=False)
    agg2 = _sc_scatter(hp2, src3, dst3, zeros)
    x2, hp3 = _mid(x1, agg2, W2_1, b2_1, W3_0, b3_0, has_residual=True)
    agg3 = _sc_scatter(hp3, src3, dst3, zeros)
    x3p = _k3(x2, agg3, W3_1, b3_1)

    pos = _pos_kernel(batch)
    g2 = _sc_head_gather(x3p, pos)
    return _head(g2, Wf0, bf0, Wf1, bf1)


# speedup vs baseline: 50.5415x; 1.0615x over previous
import functools

import jax
import jax.numpy as jnp
from jax import lax
from jax.experimental import pallas as pl
from jax.experimental.pallas import tpu as pltpu
from jax.experimental.pallas import tpu_sc as plsc

N = 100000
E = 6400000
G_PAD = 1024

_NC, _NS, _L = 2, 16, 16
_CH = 200
_GRP = 1
_EPS = E // _NS
_NIT = _EPS // _CH
_NACC = N
_RPW = 6256
_RPW_LAST = N - (_NS - 1) * _RPW


def _sc_mesh():
    return plsc.VectorSubcoreMesh(
        core_axis_name="c", subcore_axis_name="s",
        num_cores=_NC, num_subcores=_NS)


_SC_PARAMS = pltpu.CompilerParams(use_tc_tiling_on_sc=False)


def _sc_scatter_body(hp_hbm, src_hbm, dst_hbm, zeros_hbm, agg_hbm,
                     idx_s, idx_d, msg, acc, gsem, ssem, isem):
    c = lax.axis_index("c")
    s = lax.axis_index("s")

    @pl.when(s < _NS - 1)
    def _():
        pltpu.sync_copy(zeros_hbm, acc.at[pl.ds(s * _RPW, _RPW)])

    @pl.when(s == _NS - 1)
    def _():
        pltpu.sync_copy(zeros_hbm.at[pl.ds(0, _RPW_LAST)],
                        acc.at[pl.ds((_NS - 1) * _RPW, _RPW_LAST)])

    plsc.subcore_barrier()

    table = hp_hbm.at[c]

    def _load_idx_async(g, r):
        off = s * _EPS + g * _CH
        pltpu.async_copy(src_hbm.at[pl.ds(off, _CH)], idx_s.at[r, 0],
                         isem.at[r])
        pltpu.async_copy(dst_hbm.at[pl.ds(off, _CH)], idx_d.at[r, 0],
                         isem.at[r])

    def _wait_idx(g, r):
        off = s * _EPS + g * _CH
        pltpu.make_async_copy(src_hbm.at[pl.ds(off, _CH)], idx_s.at[r, 0],
                              isem.at[r]).wait()
        pltpu.make_async_copy(dst_hbm.at[pl.ds(off, _CH)], idx_d.at[r, 0],
                              isem.at[r]).wait()

    def _fire_gathers(q, r):
        pltpu.async_copy(table.at[idx_s.at[r, 0]], msg.at[q, 0], gsem.at[q])

    def _wait_gathers(q, r):
        pltpu.make_async_copy(table.at[idx_s.at[r, 0]], msg.at[q, 0],
                              gsem.at[q]).wait()

    def _fire_scatters(q, r):
        pltpu.async_copy(msg.at[q, 0], acc.at[idx_d.at[r, 0]],
                         ssem.at[q], add=True)

    def _wait_scatters(q, r):
        pltpu.make_async_copy(msg.at[q, 0], acc.at[idx_d.at[r, 0]],
                              ssem.at[q]).wait()

    def _step(g, qq):
        p = qq % 4

        @pl.when(g >= 1)
        def _():
            _wait_scatters((qq - 1) % 4, (qq - 1) % 8)

        @pl.when(g + 7 < _NIT)
        def _():
            _load_idx_async(g + 7, (qq + 7) % 8)

        @pl.when(g + 3 < _NIT)
        def _():
            _wait_idx(g + 3, (qq + 3) % 8)
            _fire_gathers((qq + 3) % 4, (qq + 3) % 8)

        _wait_gathers(p, qq % 8)
        _fire_scatters(p, qq % 8)

    for j in range(7):
        _load_idx_async(j, j)
    for g0 in range(3):
        _wait_idx(g0, g0)
        _fire_gathers(g0 % 4, g0)

    @pl.loop(0, (_NIT - 8) // 8)
    def _(t):
        for qq in range(8):
            _step(t * 8 + qq, qq)

    for qq in range(8):
        _step(_NIT - 8 + qq, qq)
    _wait_scatters(3, 7)

    plsc.subcore_barrier()

    @pl.when(s < _NS - 1)
    def _():
        pltpu.sync_copy(acc.at[pl.ds(s * _RPW, _RPW)],
                        agg_hbm.at[c, pl.ds(s * _RPW, _RPW)])

    @pl.when(s == _NS - 1)
    def _():
        pltpu.sync_copy(acc.at[pl.ds((_NS - 1) * _RPW, _RPW_LAST)],
                        agg_hbm.at[c, pl.ds((_NS - 1) * _RPW, _RPW_LAST)])


def _sc_scatter(hp, src3, dst3, zeros):
    k = pl.kernel(
        _sc_scatter_body,
        out_type=jax.ShapeDtypeStruct((_NC, N, _L), jnp.float32),
        mesh=_sc_mesh(),
        scratch_types=[
            pltpu.VMEM((8, 1, _CH), jnp.int32),
            pltpu.VMEM((8, 1, _CH), jnp.int32),
            pltpu.VMEM((4, 1, _CH, _L), jnp.float32),
            pltpu.VMEM_SHARED((_NACC, _L), jnp.float32),
            pltpu.SemaphoreType.DMA((4,)),
            pltpu.SemaphoreType.DMA((4,)),
            pltpu.SemaphoreType.DMA((8,)),
        ],
        compiler_params=_SC_PARAMS,
    )
    return k(hp, src3, dst3, zeros)


def _sc_head_gather_body(xp_hbm, ap_hbm, pos_hbm, gx_hbm, ga_hbm,
                         pidx, rows_x, rows_a, sem):
    c = lax.axis_index("c")
    s = lax.axis_index("s")
    per_w = G_PAD // _NS
    pltpu.sync_copy(pos_hbm.at[pl.ds(s * per_w, per_w)], pidx)
    cx = pltpu.async_copy(xp_hbm.at[c].at[pidx], rows_x, sem)
    ca = pltpu.async_copy(ap_hbm.at[c].at[pidx], rows_a, sem)
    cx.wait()
    ca.wait()
    pltpu.sync_copy(rows_x, gx_hbm.at[c, pl.ds(s * per_w, per_w)])
    pltpu.sync_copy(rows_a, ga_hbm.at[c, pl.ds(s * per_w, per_w)])


def _sc_head_gather(xp, ap, pos):
    per_w = G_PAD // _NS
    k = pl.kernel(
        _sc_head_gather_body,
        out_type=[
            jax.ShapeDtypeStruct((_NC, G_PAD, _L), jnp.float32),
            jax.ShapeDtypeStruct((_NC, G_PAD, _L), jnp.float32),
        ],
        mesh=_sc_mesh(),
        scratch_types=[
            pltpu.VMEM((per_w,), jnp.int32),
            pltpu.VMEM((per_w, _L), jnp.float32),
            pltpu.VMEM((per_w, _L), jnp.float32),
            pltpu.SemaphoreType.DMA,
        ],
        compiler_params=_SC_PARAMS,
    )
    return k(xp, ap, pos)


_RB = 2000


def _pack(h):
    lo = h[:, :16]
    hi = jnp.concatenate([h[:, 16:19], jnp.zeros((h.shape[0], 13), h.dtype)],
                         axis=1)
    return lo, hi


def _k0_body(x_ref, w_ref, b_ref, batch_ref, out_ref, pos_ref):
    h = jnp.dot(x_ref[...], w_ref[...],
                preferred_element_type=jnp.float32) + b_ref[...]
    lo, hi = _pack(h)
    out_ref[0] = lo
    out_ref[1] = hi

    i = pl.program_id(0)

    @pl.when(i == 0)
    def _():
        pos_ref[...] = jnp.zeros_like(pos_ref)

    b = batch_ref[0]
    gids = lax.broadcasted_iota(jnp.int32, (G_PAD, 1), 0)
    le = (b <= gids).astype(jnp.int32)
    pos_ref[...] += jnp.sum(le, axis=1, keepdims=True)

    @pl.when(i == (N // _RB) - 1)
    def _():
        pos_ref[...] = jnp.maximum(pos_ref[...] - 1, 0)


def _k0(x, W, b, batch):
    return pl.pallas_call(
        _k0_body,
        grid=(N // _RB,),
        in_specs=[
            pl.BlockSpec((_RB, 11), lambda i: (i, 0)),
            pl.BlockSpec((11, 19), lambda i: (0, 0)),
            pl.BlockSpec((1, 19), lambda i: (0, 0)),
            pl.BlockSpec((1, 1, _RB), lambda i: (i, 0, 0)),
        ],
        out_specs=[
            pl.BlockSpec((2, _RB, 16), lambda i: (0, i, 0)),
            pl.BlockSpec((G_PAD, 1), lambda i: (0, 0)),
        ],
        out_shape=[
            jax.ShapeDtypeStruct((2, N, 16), jnp.float32),
            jax.ShapeDtypeStruct((G_PAD, 1), jnp.int32),
        ],
    )(x, W, b.reshape(1, 19), batch.reshape(N // _RB, 1, _RB))


def _mid_body(has_residual, fin, x_ref, agg_ref, w1_ref, b1_ref,
              w0n_ref, b0n_ref, x1_ref, hp_ref):
    a19 = jnp.concatenate([agg_ref[0], agg_ref[1][:, :3]], axis=1)
    cat = jnp.concatenate([x_ref[...], a19], axis=1)
    y = jnp.dot(cat, w1_ref[...], preferred_element_type=jnp.float32) \
        + b1_ref[...]
    if has_residual:
        y = y + x_ref[...][:, :19]
    y = jnp.maximum(y, 0.0)
    x1_ref[...] = y
    h = jnp.dot(y, w0n_ref[...], preferred_element_type=jnp.float32) \
        + b0n_ref[...]
    lo, hi = _pack(h)
    hp_ref[0] = lo
    hp_ref[1] = hi


def _mid(x, agg, W1, b1, W0n, b0n, has_residual):
    fin = x.shape[1]
    body = functools.partial(_mid_body, has_residual, fin)
    return pl.pallas_call(
        body,
        grid=(N // _RB,),
        in_specs=[
            pl.BlockSpec((_RB, fin), lambda i: (i, 0)),
            pl.BlockSpec((2, _RB, 16), lambda i: (0, i, 0)),
            pl.BlockSpec((fin + 19, 19), lambda i: (0, 0)),
            pl.BlockSpec((1, 19), lambda i: (0, 0)),
            pl.BlockSpec((19, 19), lambda i: (0, 0)),
            pl.BlockSpec((1, 19), lambda i: (0, 0)),
        ],
        out_specs=[
            pl.BlockSpec((_RB, 19), lambda i: (i, 0)),
            pl.BlockSpec((2, _RB, 16), lambda i: (0, i, 0)),
        ],
        out_shape=[
            jax.ShapeDtypeStruct((N, 19), jnp.float32),
            jax.ShapeDtypeStruct((2, N, 16), jnp.float32),
        ],
    )(x, agg, W1, b1.reshape(1, 19), W0n, b0n.reshape(1, 19))


def _mid_packed_body(x_ref, agg_ref, w1_ref, b1_ref,
                     w0n_ref, b0n_ref, xp_ref, hp_ref):
    a19 = jnp.concatenate([agg_ref[0], agg_ref[1][:, :3]], axis=1)
    cat = jnp.concatenate([x_ref[...], a19], axis=1)
    y = jnp.dot(cat, w1_ref[...], preferred_element_type=jnp.float32) \
        + b1_ref[...]
    y = jnp.maximum(y + x_ref[...], 0.0)
    lo, hi = _pack(y)
    xp_ref[0] = lo
    xp_ref[1] = hi
    h = jnp.dot(y, w0n_ref[...], preferred_element_type=jnp.float32) \
        + b0n_ref[...]
    lo, hi = _pack(h)
    hp_ref[0] = lo
    hp_ref[1] = hi


def _mid_packed(x, agg, W1, b1, W0n, b0n):
    return pl.pallas_call(
        _mid_packed_body,
        grid=(N // _RB,),
        in_specs=[
            pl.BlockSpec((_RB, 19), lambda i: (i, 0)),
            pl.BlockSpec((2, _RB, 16), lambda i: (0, i, 0)),
            pl.BlockSpec((38, 19), lambda i: (0, 0)),
            pl.BlockSpec((1, 19), lambda i: (0, 0)),
            pl.BlockSpec((19, 19), lambda i: (0, 0)),
            pl.BlockSpec((1, 19), lambda i: (0, 0)),
        ],
        out_specs=[
            pl.BlockSpec((2, _RB, 16), lambda i: (0, i, 0)),
            pl.BlockSpec((2, _RB, 16), lambda i: (0, i, 0)),
        ],
        out_shape=[
            jax.ShapeDtypeStruct((2, N, 16), jnp.float32),
            jax.ShapeDtypeStruct((2, N, 16), jnp.float32),
        ],
    )(x, agg, W1, b1.reshape(1, 19), W0n, b0n.reshape(1, 19))


def _head_body(gx_ref, ga_ref, w31_ref, b31_ref, w0_ref, b0_ref,
               w1_ref, b1_ref, out_ref):
    x2g = jnp.concatenate([gx_ref[0, :1000, :], gx_ref[1, :1000, :3]], axis=1)
    a3g = jnp.concatenate([ga_ref[0, :1000, :], ga_ref[1, :1000, :3]], axis=1)
    cat = jnp.concatenate([x2g, a3g], axis=1)
    y = jnp.dot(cat, w31_ref[...], preferred_element_type=jnp.float32) \
        + b31_ref[...]
    y = jnp.maximum(y + x2g, 0.0)
    t = jnp.maximum(
        jnp.dot(y, w0_ref[...], preferred_element_type=jnp.float32)
        + b0_ref[...], 0.0)
    out_ref[...] = jnp.maximum(
        jnp.dot(t, w1_ref[...], preferred_element_type=jnp.float32)
        + b1_ref[...], 0.0)


def _head(gx, ga, W31, b31, Wf0, bf0, Wf1, bf1):
    return pl.pallas_call(
        _head_body,
        in_specs=[
            pl.BlockSpec((2, G_PAD, 16), lambda: (0, 0, 0)),
            pl.BlockSpec((2, G_PAD, 16), lambda: (0, 0, 0)),
            pl.BlockSpec((38, 19), lambda: (0, 0)),
            pl.BlockSpec((1, 19), lambda: (0, 0)),
            pl.BlockSpec((19, 10), lambda: (0, 0)),
            pl.BlockSpec((1, 10), lambda: (0, 0)),
            pl.BlockSpec((10, 1), lambda: (0, 0)),
            pl.BlockSpec((1, 1), lambda: (0, 0)),
        ],
        out_specs=pl.BlockSpec((1000, 1), lambda: (0, 0)),
        out_shape=jax.ShapeDtypeStruct((1000, 1), jnp.float32),
        grid=(),
    )(gx, ga, W31, b31.reshape(1, 19), Wf0, bf0.reshape(1, 10),
      Wf1, bf1.reshape(1, 1))


def kernel(x, edge_index, batch, W1_0, b1_0, W1_1, b1_1, W2_0, b2_0,
           W2_1, b2_1, W3_0, b3_0, W3_1, b3_1, Wf0, bf0, Wf1, bf1):
    src3 = edge_index[0]
    dst3 = edge_index[1]
    zeros = jnp.zeros((_RPW, _L), jnp.float32)

    hp1, pos = _k0(x, W1_0, b1_0, batch)
    agg1 = _sc_scatter(hp1, src3, dst3, zeros)
    x1, hp2 = _mid(x, agg1, W1_1, b1_1, W2_0, b2_0, has_residual=False)
    agg2 = _sc_scatter(hp2, src3, dst3, zeros)
    x2p, hp3 = _mid_packed(x1, agg2, W2_1, b2_1, W3_0, b3_0)
    agg3 = _sc_scatter(hp3, src3, dst3, zeros)

    gx, ga = _sc_head_gather(x2p, agg3, pos.reshape(G_PAD))
    return _head(gx, ga, W3_1, b3_1, Wf0, bf0, Wf1, bf1)
